# baseline (speedup 1.0000x reference)
import functools

import jax
import jax.numpy as jnp
from jax import lax
from jax.experimental import pallas as pl
from jax.experimental.pallas import tpu as pltpu
from jax.experimental.pallas import tpu_sc as plsc

N_NODES = 100000
D = 256
HIDDEN = 256
B = 1024
FAN = 10
N1 = B + B * FAN
N0 = N1 + N1 * FAN
E0 = N1 * FAN
E1 = B * FAN
CPAD = 128

NC = 2
NS = 16
NW = NC * NS

DPT0 = N1 // NW
EPT0 = DPT0 * FAN
CH0 = 16
NCH0 = DPT0 // CH0
EPC0 = CH0 * FAN
G0 = 80
NGID = EPT0 // G0

DPT1 = B // NW
EPT1 = DPT1 * FAN
G1 = 80
NG1 = EPT1 // G1


def _acc_rows(rows_ref, acc_ref, d, ncol):
    base = d * FAN
    for c in range(ncol // 16):
        sl = pl.ds(c * 16, 16)
        v = rows_ref[base, sl]
        for r in range(1, FAN):
            v = v + rows_ref[base + r, sl]
        acc_ref[d, sl] = v


def _mesh():
    return plsc.VectorSubcoreMesh(
        core_axis_name="c", subcore_axis_name="s",
        num_cores=NC, num_subcores=NS)


@functools.partial(
    pl.kernel,
    out_type=(
        jax.ShapeDtypeStruct((N1, D), jnp.float32),
        jax.ShapeDtypeStruct((N1, D), jnp.float32),
    ),
    mesh=_mesh(),
    scratch_types=[
        pltpu.VMEM((EPT0,), jnp.int32),
        pltpu.VMEM((EPT0,), jnp.int32),
        pltpu.VMEM((DPT0,), jnp.int32),
        pltpu.VMEM((2, EPC0, D), jnp.float32),
        pltpu.VMEM((128, D), jnp.float32),
        pltpu.VMEM((2, CH0, D), jnp.float32),
        pltpu.SemaphoreType.DMA,
        pltpu.SemaphoreType.DMA,
        pltpu.SemaphoreType.DMA,
        pltpu.SemaphoreType.DMA,
    ],
)
def _sc_layer0(node_feat, gids0, col0, sum0, xt,
               colbuf, gidx, tgid, rows, xtbuf, acc, semA, semB, semI, semO):
    wid = lax.axis_index("s") * NC + lax.axis_index("c")
    ebase = wid * EPT0
    dbase = wid * DPT0
    sems = (semA, semB)

    pltpu.sync_copy(col0.at[pl.ds(ebase, EPT0)], colbuf)
    for g in range(NGID):
        sl = pl.ds(g * G0, G0)
        pltpu.async_copy(gids0.at[colbuf.at[sl]], gidx.at[sl], semI)

    pltpu.sync_copy(gids0.at[pl.ds(dbase, DPT0)], tgid)
    tchunks = ((0, 128), (128, 128), (256, 96))

    def tfire(off, n):
        pltpu.async_copy(
            node_feat.at[tgid.at[pl.ds(off, n)]],
            xtbuf.at[pl.ds(0, n)], semB)

    def tdrain(off, n):
        pltpu.make_async_copy(
            node_feat.at[tgid.at[pl.ds(off, n)]],
            xtbuf.at[pl.ds(0, n)], semB).wait()
        pltpu.sync_copy(xtbuf.at[pl.ds(0, n)], xt.at[pl.ds(dbase + off, n)])

    tfire(*tchunks[0])
    tdrain(*tchunks[0])
    tfire(*tchunks[1])
    tdrain(*tchunks[1])
    tfire(*tchunks[2])
    tdrain(*tchunks[2])

    for g in range(NGID):
        sl = pl.ds(g * G0, G0)
        pltpu.make_async_copy(gids0.at[colbuf.at[sl]], gidx.at[sl], semI).wait()

    def fire(j, b):
        eoff = j * EPC0
        pltpu.async_copy(
            node_feat.at[gidx.at[pl.ds(eoff, G0)]],
            rows.at[b].at[pl.ds(0, G0)], sems[b])
        pltpu.async_copy(
            node_feat.at[gidx.at[pl.ds(eoff + G0, G0)]],
            rows.at[b].at[pl.ds(G0, G0)], sems[b])

    def drain(j, b):
        eoff = j * EPC0
        pltpu.make_async_copy(
            node_feat.at[gidx.at[pl.ds(eoff, G0)]],
            rows.at[b].at[pl.ds(0, G0)], sems[b]).wait()
        pltpu.make_async_copy(
            node_feat.at[gidx.at[pl.ds(eoff + G0, G0)]],
            rows.at[b].at[pl.ds(G0, G0)], sems[b]).wait()

    def ofire(j, b):
        pltpu.async_copy(acc.at[b], sum0.at[pl.ds(dbase + j * CH0, CH0)], semO)

    def odrain(j, b):
        pltpu.make_async_copy(
            acc.at[b], sum0.at[pl.ds(dbase + j * CH0, CH0)], semO).wait()

    fire(0, 0)

    def outer(i2, carry):
        for b in range(2):
            j = i2 * 2 + b

            @pl.when(j + 1 < NCH0)
            def _():
                fire(j + 1, 1 - b)

            drain(j, b)

            @pl.when(j >= 2)
            def _():
                odrain(j - 2, b)

            rows_b = rows.at[b]
            acc_b = acc.at[b]

            def dst_body(d, carry2):
                _acc_rows(rows_b, acc_b, d, D)
                return carry2
            lax.fori_loop(0, CH0, dst_body, 0, unroll=4)
            ofire(j, b)
        return carry
    lax.fori_loop(0, NCH0 // 2, outer, 0, unroll=False)
    odrain(NCH0 - 2, 0)
    odrain(NCH0 - 1, 1)


@functools.partial(
    pl.kernel,
    out_type=jax.ShapeDtypeStruct((B, CPAD), jnp.float32),
    mesh=_mesh(),
    scratch_types=[
        pltpu.VMEM((EPT1,), jnp.int32),
        pltpu.VMEM((2, EPC0, CPAD), jnp.float32),
        pltpu.VMEM((DPT1, CPAD), jnp.float32),
        pltpu.VMEM((CPAD,), jnp.float32),
        pltpu.VMEM((DPT1, CPAD), jnp.float32),
        pltpu.SemaphoreType.DMA,
        pltpu.SemaphoreType.DMA,
    ],
)
def _sc_layer1(z, selfz, b1p, col1, out,
               colbuf, rows, selfv, bv, outv, semA, semB):
    wid = lax.axis_index("s") * NC + lax.axis_index("c")
    ebase = wid * EPT1
    dbase = wid * DPT1
    sems = (semA, semB)

    pltpu.sync_copy(col1.at[pl.ds(ebase, EPT1)], colbuf)
    pltpu.sync_copy(selfz.at[pl.ds(dbase, DPT1)], selfv)
    pltpu.sync_copy(b1p, bv)

    def fire(j, b):
        eoff = j * EPC0
        pltpu.async_copy(
            z.at[colbuf.at[pl.ds(eoff, G1)]],
            rows.at[b].at[pl.ds(0, G1)], sems[b])
        pltpu.async_copy(
            z.at[colbuf.at[pl.ds(eoff + G1, G1)]],
            rows.at[b].at[pl.ds(G1, G1)], sems[b])

    def drain(j, b):
        eoff = j * EPC0
        pltpu.make_async_copy(
            z.at[colbuf.at[pl.ds(eoff, G1)]],
            rows.at[b].at[pl.ds(0, G1)], sems[b]).wait()
        pltpu.make_async_copy(
            z.at[colbuf.at[pl.ds(eoff + G1, G1)]],
            rows.at[b].at[pl.ds(G1, G1)], sems[b]).wait()

    fire(0, 0)
    fire(1, 1)
    inv_fan = 1.0 / FAN
    for jj in range(2):
        drain(jj, jj)
        rows_b = rows.at[jj]
        doff = jj * CH0

        def dst_body(d, carry):
            base = d * FAN
            for c in range(CPAD // 16):
                sl = pl.ds(c * 16, 16)
                v = rows_b[base, sl]
                for r in range(1, FAN):
                    v = v + rows_b[base + r, sl]
                outv[doff + d, sl] = (selfv[doff + d, sl] + v * inv_fan
                                      + bv[sl])
            return carry
        lax.fori_loop(0, CH0, dst_body, 0, unroll=2)
    pltpu.sync_copy(outv, out.at[pl.ds(dbase, DPT1)])


def _tc_fused(xt, sum0, W_self0, W_neigh0, b0, Wcat1):
    BLK = 512

    def body(xt_ref, s0_ref, ws_ref, wn_ref, b_ref, wc_ref, z_ref, sz_ref):
        xtb = xt_ref[...].astype(jnp.bfloat16)
        mean = (s0_ref[...] * (1.0 / FAN)).astype(jnp.bfloat16)
        h = jnp.maximum(
            jnp.dot(xtb, ws_ref[...], preferred_element_type=jnp.float32)
            + jnp.dot(mean, wn_ref[...], preferred_element_type=jnp.float32)
            + b_ref[...], 0.0)
        zsz = jnp.dot(h.astype(jnp.bfloat16), wc_ref[...],
                      preferred_element_type=jnp.float32)
        z_ref[...] = zsz[:, :CPAD]
        sz_ref[...] = zsz[:, CPAD:]

    return pl.pallas_call(
        body,
        grid=(N1 // BLK,),
        in_specs=[
            pl.BlockSpec((BLK, D), lambda i: (i, 0)),
            pl.BlockSpec((BLK, D), lambda i: (i, 0)),
            pl.BlockSpec((D, HIDDEN), lambda i: (0, 0)),
            pl.BlockSpec((D, HIDDEN), lambda i: (0, 0)),
            pl.BlockSpec((1, HIDDEN), lambda i: (0, 0)),
            pl.BlockSpec((HIDDEN, 2 * CPAD), lambda i: (0, 0)),
        ],
        out_specs=[
            pl.BlockSpec((BLK, CPAD), lambda i: (i, 0)),
            pl.BlockSpec((BLK, CPAD), lambda i: (i, 0)),
        ],
        out_shape=[
            jax.ShapeDtypeStruct((N1, CPAD), jnp.float32),
            jax.ShapeDtypeStruct((N1, CPAD), jnp.float32),
        ],
    )(xt, sum0, W_self0, W_neigh0, b0, Wcat1)


def kernel(node_feat, gids0, csr_row_ptr0, csr_col_ind0, csr_row_ptr1,
           csr_col_ind1, W_self0, W_neigh0, b0, W_self1, W_neigh1, b1):
    del csr_row_ptr0, csr_row_ptr1
    ncls = W_self1.shape[1]
    pad = CPAD - ncls
    Wsp = jnp.pad(W_self1, ((0, 0), (0, pad)))
    Wnp = jnp.pad(W_neigh1, ((0, 0), (0, pad)))
    b1p = jnp.pad(b1, (0, pad))
    Wcat1 = jnp.concatenate([Wnp, Wsp], axis=1).astype(jnp.bfloat16)

    sum0, xt = _sc_layer0(node_feat, gids0, csr_col_ind0)
    z, selfz = _tc_fused(xt, sum0, W_self0.astype(jnp.bfloat16),
                         W_neigh0.astype(jnp.bfloat16),
                         b0.reshape(1, HIDDEN), Wcat1)
    out = _sc_layer1(z, selfz, b1p, csr_col_ind1)
    return out[:, :ncls]


# speedup vs baseline: 1.5282x; 1.5282x over previous
import functools

import jax
import jax.numpy as jnp
from jax import lax
from jax.experimental import pallas as pl
from jax.experimental.pallas import tpu as pltpu
from jax.experimental.pallas import tpu_sc as plsc

N_NODES = 100000
D = 256
HIDDEN = 256
B = 1024
FAN = 10
N1 = B + B * FAN
N0 = N1 + N1 * FAN
E0 = N1 * FAN
E1 = B * FAN
CPAD = 128

NC = 2
NS = 16
NW = NC * NS

DPT0 = N1 // NW
EPT0 = DPT0 * FAN
CH0 = 16
NCH0 = DPT0 // CH0
EPC0 = CH0 * FAN
G0 = 80
NGID = EPT0 // G0

DPT1 = B // NW
EPT1 = DPT1 * FAN
G1 = 80
NG1 = EPT1 // G1


def _acc_rows(rows_ref, acc_ref, d, ncol):
    base = d * FAN
    for c in range(ncol // 16):
        sl = pl.ds(c * 16, 16)
        v = rows_ref[base, sl]
        for r in range(1, FAN):
            v = v + rows_ref[base + r, sl]
        acc_ref[d, sl] = v


def _mesh():
    return plsc.VectorSubcoreMesh(
        core_axis_name="c", subcore_axis_name="s",
        num_cores=NC, num_subcores=NS)


@functools.partial(
    pl.kernel,
    out_type=(
        jax.ShapeDtypeStruct((N1, D), jnp.float32),
        jax.ShapeDtypeStruct((N1, D), jnp.float32),
    ),
    mesh=_mesh(),
    scratch_types=[
        pltpu.VMEM((EPT0,), jnp.int32),
        pltpu.VMEM((EPT0,), jnp.int32),
        pltpu.VMEM((DPT0,), jnp.int32),
        pltpu.VMEM((2, EPC0, D), jnp.float32),
        pltpu.VMEM((128, D), jnp.float32),
        pltpu.VMEM((2, CH0, D), jnp.float32),
        pltpu.SemaphoreType.DMA,
        pltpu.SemaphoreType.DMA,
        pltpu.SemaphoreType.DMA,
        pltpu.SemaphoreType.DMA,
    ],
)
def _sc_layer0(node_feat, gids0, col0, sum0, xt,
               colbuf, gidx, tgid, rows, xtbuf, acc, semA, semB, semI, semO):
    wid = lax.axis_index("s") * NC + lax.axis_index("c")
    ebase = wid * EPT0
    dbase = wid * DPT0
    sems = (semA, semB)

    pltpu.sync_copy(col0.at[pl.ds(ebase, EPT0)], colbuf)
    for g in range(NGID):
        sl = pl.ds(g * G0, G0)
        pltpu.async_copy(gids0.at[colbuf.at[sl]], gidx.at[sl], semI)

    pltpu.sync_copy(gids0.at[pl.ds(dbase, DPT0)], tgid)
    tchunks = ((0, 128), (128, 128), (256, 96))

    def tfire(off, n):
        pltpu.async_copy(
            node_feat.at[tgid.at[pl.ds(off, n)]],
            xtbuf.at[pl.ds(0, n)], semB)

    def tdrain(off, n):
        pltpu.make_async_copy(
            node_feat.at[tgid.at[pl.ds(off, n)]],
            xtbuf.at[pl.ds(0, n)], semB).wait()
        pltpu.sync_copy(xtbuf.at[pl.ds(0, n)], xt.at[pl.ds(dbase + off, n)])

    tfire(*tchunks[0])
    tdrain(*tchunks[0])
    tfire(*tchunks[1])
    tdrain(*tchunks[1])
    tfire(*tchunks[2])
    tdrain(*tchunks[2])

    for g in range(NGID):
        sl = pl.ds(g * G0, G0)
        pltpu.make_async_copy(gids0.at[colbuf.at[sl]], gidx.at[sl], semI).wait()

    def fire(j, b):
        eoff = j * EPC0
        pltpu.async_copy(
            node_feat.at[gidx.at[pl.ds(eoff, G0)]],
            rows.at[b].at[pl.ds(0, G0)], sems[b])
        pltpu.async_copy(
            node_feat.at[gidx.at[pl.ds(eoff + G0, G0)]],
            rows.at[b].at[pl.ds(G0, G0)], sems[b])

    def drain(j, b):
        eoff = j * EPC0
        pltpu.make_async_copy(
            node_feat.at[gidx.at[pl.ds(eoff, G0)]],
            rows.at[b].at[pl.ds(0, G0)], sems[b]).wait()
        pltpu.make_async_copy(
            node_feat.at[gidx.at[pl.ds(eoff + G0, G0)]],
            rows.at[b].at[pl.ds(G0, G0)], sems[b]).wait()

    def ofire(j, b):
        pltpu.async_copy(acc.at[b], sum0.at[pl.ds(dbase + j * CH0, CH0)], semO)

    def odrain(j, b):
        pltpu.make_async_copy(
            acc.at[b], sum0.at[pl.ds(dbase + j * CH0, CH0)], semO).wait()

    fire(0, 0)

    def outer(i2, carry):
        for b in range(2):
            j = i2 * 2 + b

            @pl.when(j + 1 < NCH0)
            def _():
                fire(j + 1, 1 - b)

            drain(j, b)

            @pl.when(j >= 2)
            def _():
                odrain(j - 2, b)

            rows_b = rows.at[b]
            acc_b = acc.at[b]

            def dst_body(d, carry2):
                _acc_rows(rows_b, acc_b, d, D)
                return carry2
            lax.fori_loop(0, CH0, dst_body, 0, unroll=2)
            ofire(j, b)
        return carry
    lax.fori_loop(0, NCH0 // 2, outer, 0, unroll=False)
    odrain(NCH0 - 2, 0)
    odrain(NCH0 - 1, 1)


@functools.partial(
    pl.kernel,
    out_type=jax.ShapeDtypeStruct((B, CPAD), jnp.float32),
    mesh=_mesh(),
    scratch_types=[
        pltpu.VMEM((EPT1,), jnp.int32),
        pltpu.VMEM((2, EPC0, CPAD), jnp.float32),
        pltpu.VMEM((DPT1, CPAD), jnp.float32),
        pltpu.VMEM((CPAD,), jnp.float32),
        pltpu.VMEM((DPT1, CPAD), jnp.float32),
        pltpu.SemaphoreType.DMA,
        pltpu.SemaphoreType.DMA,
    ],
)
def _sc_layer1(z, selfz, b1p, col1, out,
               colbuf, rows, selfv, bv, outv, semA, semB):
    wid = lax.axis_index("s") * NC + lax.axis_index("c")
    ebase = wid * EPT1
    dbase = wid * DPT1
    sems = (semA, semB)

    pltpu.sync_copy(col1.at[pl.ds(ebase, EPT1)], colbuf)
    pltpu.sync_copy(selfz.at[pl.ds(dbase, DPT1)], selfv)
    pltpu.sync_copy(b1p, bv)

    def fire(j, b):
        eoff = j * EPC0
        pltpu.async_copy(
            z.at[colbuf.at[pl.ds(eoff, G1)]],
            rows.at[b].at[pl.ds(0, G1)], sems[b])
        pltpu.async_copy(
            z.at[colbuf.at[pl.ds(eoff + G1, G1)]],
            rows.at[b].at[pl.ds(G1, G1)], sems[b])

    def drain(j, b):
        eoff = j * EPC0
        pltpu.make_async_copy(
            z.at[colbuf.at[pl.ds(eoff, G1)]],
            rows.at[b].at[pl.ds(0, G1)], sems[b]).wait()
        pltpu.make_async_copy(
            z.at[colbuf.at[pl.ds(eoff + G1, G1)]],
            rows.at[b].at[pl.ds(G1, G1)], sems[b]).wait()

    fire(0, 0)
    fire(1, 1)
    inv_fan = 1.0 / FAN
    for jj in range(2):
        drain(jj, jj)
        rows_b = rows.at[jj]
        doff = jj * CH0

        def dst_body(d, carry):
            base = d * FAN
            for c in range(CPAD // 16):
                sl = pl.ds(c * 16, 16)
                v = rows_b[base, sl]
                for r in range(1, FAN):
                    v = v + rows_b[base + r, sl]
                outv[doff + d, sl] = (selfv[doff + d, sl] + v * inv_fan
                                      + bv[sl])
            return carry
        lax.fori_loop(0, CH0, dst_body, 0, unroll=2)
    pltpu.sync_copy(outv, out.at[pl.ds(dbase, DPT1)])


def _tc_fused(xt, sum0, W_self0, W_neigh0, b0, Wcat1):
    BLK = 2048

    def body(xt_ref, s0_ref, ws_ref, wn_ref, b_ref, wc_ref, z_ref, sz_ref):
        xtb = xt_ref[...].astype(jnp.bfloat16)
        mean = (s0_ref[...] * (1.0 / FAN)).astype(jnp.bfloat16)
        h = jnp.maximum(
            jnp.dot(xtb, ws_ref[...], preferred_element_type=jnp.float32)
            + jnp.dot(mean, wn_ref[...], preferred_element_type=jnp.float32)
            + b_ref[...], 0.0)
        zsz = jnp.dot(h.astype(jnp.bfloat16), wc_ref[...],
                      preferred_element_type=jnp.float32)
        z_ref[...] = zsz[:, :CPAD]
        sz_ref[...] = zsz[:, CPAD:]

    return pl.pallas_call(
        body,
        grid=(N1 // BLK,),
        in_specs=[
            pl.BlockSpec((BLK, D), lambda i: (i, 0)),
            pl.BlockSpec((BLK, D), lambda i: (i, 0)),
            pl.BlockSpec((D, HIDDEN), lambda i: (0, 0)),
            pl.BlockSpec((D, HIDDEN), lambda i: (0, 0)),
            pl.BlockSpec((1, HIDDEN), lambda i: (0, 0)),
            pl.BlockSpec((HIDDEN, 2 * CPAD), lambda i: (0, 0)),
        ],
        out_specs=[
            pl.BlockSpec((BLK, CPAD), lambda i: (i, 0)),
            pl.BlockSpec((BLK, CPAD), lambda i: (i, 0)),
        ],
        out_shape=[
            jax.ShapeDtypeStruct((N1, CPAD), jnp.float32),
            jax.ShapeDtypeStruct((N1, CPAD), jnp.float32),
        ],
    )(xt, sum0, W_self0, W_neigh0, b0, Wcat1)


def kernel(node_feat, gids0, csr_row_ptr0, csr_col_ind0, csr_row_ptr1,
           csr_col_ind1, W_self0, W_neigh0, b0, W_self1, W_neigh1, b1):
    del csr_row_ptr0, csr_row_ptr1
    ncls = W_self1.shape[1]
    pad = CPAD - ncls
    Wsp = jnp.pad(W_self1, ((0, 0), (0, pad)))
    Wnp = jnp.pad(W_neigh1, ((0, 0), (0, pad)))
    b1p = jnp.pad(b1, (0, pad))
    Wcat1 = jnp.concatenate([Wnp, Wsp], axis=1).astype(jnp.bfloat16)

    sum0, xt = _sc_layer0(node_feat, gids0, csr_col_ind0)
    z, selfz = _tc_fused(xt, sum0, W_self0.astype(jnp.bfloat16),
                         W_neigh0.astype(jnp.bfloat16),
                         b0.reshape(1, HIDDEN), Wcat1)
    out = _sc_layer1(z, selfz, b1p, csr_col_ind1)
    return out[:, :ncls]
